# baseline (speedup 1.0000x reference)
import functools

import jax
import jax.numpy as jnp
from jax.experimental import pallas as pl
from jax.experimental.pallas import tpu as pltpu

BETA = 0.25


def _vq_kernel(x_ref, w_ref, loss_ref, xq_ref, *, scale):
    i = pl.program_id(0)
    last = pl.num_programs(0) - 1
    xq_ref[...] = x_ref[...]
    nb = x_ref.shape[0]
    w = w_ref[...]
    wsq = jnp.sum(w * w, axis=1, keepdims=True)
    wb = w.astype(jnp.bfloat16)
    partial = jnp.float32(0.0)
    for j in range(nb):
        xj = x_ref[j]
        scores = jax.lax.dot_general(
            wb, xj.astype(jnp.bfloat16),
            dimension_numbers=(((1,), (0,)), ((), ())),
            preferred_element_type=jnp.float32)
        dmin = jnp.min(wsq - 2.0 * scores, axis=0)
        partial += jnp.sum(dmin) + jnp.sum(xj * xj)
    total = jnp.where(i == 0, 0.0, loss_ref[0, 0]) + partial
    loss_ref[...] = jnp.where(i == last, total * scale, total).reshape(1, 1)


def kernel(x, W):
    b, c, h, w = x.shape
    pos = h * w
    codes, dim = W.shape
    bb = 4
    xr = x.reshape(b, c, pos)
    scale = (1.0 + BETA) / float(x.size)
    body = functools.partial(_vq_kernel, scale=scale)
    loss, xq = pl.pallas_call(
        body,
        grid=(b // bb,),
        in_specs=[
            pl.BlockSpec((bb, c, pos), lambda i: (i, 0, 0)),
            pl.BlockSpec((codes, dim), lambda i: (0, 0)),
        ],
        out_specs=[
            pl.BlockSpec((1, 1), lambda i: (0, 0)),
            pl.BlockSpec((bb, c, pos), lambda i: (i, 0, 0)),
        ],
        out_shape=[
            jax.ShapeDtypeStruct((1, 1), jnp.float32),
            jax.ShapeDtypeStruct((b, c, pos), jnp.float32),
        ],
        compiler_params=pltpu.CompilerParams(
            vmem_limit_bytes=100 * 1024 * 1024),
    )(xr, W)
    return (xq.reshape(b, c, h, w), loss[0, 0])
